# baseline (speedup 1.0000x reference)
import jax
import jax.numpy as jnp
from jax import lax
from jax.experimental import pallas as pl
from jax.experimental.pallas import tpu as pltpu
from jax.experimental.pallas import tpu_sc as plsc

U = 10000
UP = 10240
D = 128
E = 320000
B = 4096

NC = 2
NS = 16
NW = NC * NS
PW = E // NW
CH = 40
NCH = PW // CH
NBUF = 5
NG = NCH // NBUF
SBYTES = CH * 128 * 4
RT = UP // NS
ZR = 128
BW = B // NW

_f32 = jnp.float32

_mesh = plsc.VectorSubcoreMesh(
    core_axis_name="c", subcore_axis_name="s", num_cores=NC, num_subcores=NS
)


def _agg_body(tbl, src, dst, zrows, acc_out, *refs):
    ids = (refs[0:NBUF], refs[NBUF:2 * NBUF])
    idd = (refs[2 * NBUF:3 * NBUF], refs[3 * NBUF:4 * NBUF])
    rows, stage, acc_sh, gsem, ssem, isem = refs[4 * NBUF:]
    cid = lax.axis_index("c")
    sid = lax.axis_index("s")
    wid = cid * NS + sid
    r0 = sid * RT
    pltpu.sync_copy(zrows, stage)
    for t in range(RT // ZR):
        pltpu.sync_copy(stage, acc_sh.at[pl.ds(r0 + t * ZR, ZR)])
    plsc.subcore_barrier()

    base = wid * PW

    def fire_idx(g, p):
        for b in range(NBUF):
            off = base + (g * NBUF + b) * CH
            pltpu.async_copy(src.at[pl.ds(off, CH)], ids[p][b], isem)
            pltpu.async_copy(dst.at[pl.ds(off, CH)], idd[p][b], isem)

    fire_idx(0, 0)
    fire_idx(1, 1)

    def super_group(k, carry):
        for p in range(2):
            g = 2 * k + p
            for b in range(NBUF):
                pltpu.make_async_copy(
                    src.at[pl.ds(0, CH)], ids[p][b], isem).wait()
                pltpu.make_async_copy(
                    src.at[pl.ds(0, CH)], idd[p][b], isem).wait()
            @pl.when(g > 0)
            def _():
                for b in range(NBUF):
                    pltpu.make_async_copy(
                        tbl.at[pl.ds(0, CH)], rows.at[b], ssem).wait()
            descs = []
            for b in range(NBUF):
                descs.append(
                    pltpu.async_copy(tbl.at[ids[p][b]], rows.at[b], gsem))
            for b in range(NBUF):
                descs[b].wait()
                pltpu.async_copy(rows.at[b], acc_sh.at[idd[p][b]], ssem,
                                 add=True)
            @pl.when(g + 2 < NG)
            def _():
                fire_idx(g + 2, p)
        return carry

    lax.fori_loop(0, NG // 2, super_group, 0)
    for b in range(NBUF):
        pltpu.make_async_copy(tbl.at[pl.ds(0, CH)], rows.at[b], ssem).wait()
    plsc.subcore_barrier()
    for t in range(RT // ZR):
        sl = pl.ds(r0 + t * ZR, ZR)
        pltpu.sync_copy(acc_sh.at[sl], stage)
        pltpu.sync_copy(stage, acc_out.at[cid, sl])


_agg = pl.kernel(
    _agg_body,
    out_type=jax.ShapeDtypeStruct((NC, UP, D), _f32),
    mesh=_mesh,
    scratch_types=[
        *([pltpu.VMEM((CH,), jnp.int32)] * (4 * NBUF)),
        pltpu.VMEM((NBUF, CH, D), _f32),
        pltpu.VMEM((ZR, D), _f32),
        pltpu.VMEM_SHARED((UP, D), _f32),
        pltpu.SemaphoreType.DMA,
        pltpu.SemaphoreType.DMA,
        pltpu.SemaphoreType.DMA,
    ],
)


def _batch_body(fu, fi, users, items, lu_out, li_out, idx_v, rows_v, sem):
    cid = lax.axis_index("c")
    sid = lax.axis_index("s")
    wid = cid * NS + sid
    b0 = wid * BW
    pltpu.sync_copy(users.at[pl.ds(b0, BW)], idx_v)
    pltpu.async_copy(fu.at[idx_v], rows_v, sem).wait()
    pltpu.sync_copy(rows_v, lu_out.at[pl.ds(b0, BW)])
    pltpu.sync_copy(items.at[pl.ds(b0, BW)], idx_v)
    pltpu.async_copy(fi.at[idx_v], rows_v, sem).wait()
    pltpu.sync_copy(rows_v, li_out.at[pl.ds(b0, BW)])


_batch_gather = pl.kernel(
    _batch_body,
    out_type=(
        jax.ShapeDtypeStruct((B, D), _f32),
        jax.ShapeDtypeStruct((B, D), _f32),
    ),
    mesh=_mesh,
    scratch_types=[
        pltpu.VMEM((BW,), jnp.int32),
        pltpu.VMEM((BW, D), _f32),
        pltpu.SemaphoreType.DMA,
    ],
)


RB = 1024


SHIFT = 8.0


def _norm_terms(acc_ref, self_unshifted):
    t = acc_ref[0] + acc_ref[1]
    d = jnp.round(t[:, 0:1] * (1.0 / SHIFT))
    agg = t - SHIFT * d
    sw = 1.0 - d / (d + 1e-8)
    den = jnp.maximum(d, 1.0)
    return sw * self_unshifted + agg / den


def _comb_kernel(acc_a, acc_b, self_a, self_b, out_a, out_b):
    out_a[...] = _norm_terms(acc_a, self_a[...]) + SHIFT
    out_b[...] = _norm_terms(acc_b, self_b[...]) + SHIFT


def _comb_fin_kernel(acc_a, acc_b, self_a, self_b, base_a, base_b,
                     fin_a, fin_b):
    su_a = self_a[...] - SHIFT
    su_b = self_b[...] - SHIFT
    fin_a[...] = base_a[...] + su_a + _norm_terms(acc_a, su_a)
    fin_b[...] = base_b[...] + su_b + _norm_terms(acc_b, su_b)


_acc_spec = pl.BlockSpec((NC, RB, D), lambda i: (0, i, 0))
_tbl_spec = pl.BlockSpec((RB, D), lambda i: (i, 0))

_comb = pl.pallas_call(
    _comb_kernel,
    out_shape=(
        jax.ShapeDtypeStruct((UP, D), _f32),
        jax.ShapeDtypeStruct((UP, D), _f32),
    ),
    grid=(UP // RB,),
    in_specs=[_acc_spec, _acc_spec, _tbl_spec, _tbl_spec],
    out_specs=(_tbl_spec, _tbl_spec),
)

_comb_fin = pl.pallas_call(
    _comb_fin_kernel,
    out_shape=(
        jax.ShapeDtypeStruct((UP, D), _f32),
        jax.ShapeDtypeStruct((UP, D), _f32),
    ),
    grid=(UP // RB,),
    in_specs=[_acc_spec, _acc_spec, _tbl_spec, _tbl_spec,
              _tbl_spec, _tbl_spec],
    out_specs=(_tbl_spec, _tbl_spec),
)


PB = 512


def _pred_kernel(lu_ref, li_ref, out_ref):
    s = jnp.sum(lu_ref[...] * li_ref[...], axis=1)
    out_ref[...] = jax.nn.sigmoid(s)


_pred = pl.pallas_call(
    _pred_kernel,
    out_shape=jax.ShapeDtypeStruct((B,), _f32),
    grid=(B // PB,),
    in_specs=[
        pl.BlockSpec((PB, D), lambda i: (i, 0)),
        pl.BlockSpec((PB, D), lambda i: (i, 0)),
    ],
    out_specs=pl.BlockSpec((PB,), lambda i: (i,)),
)


def kernel(user_emb, item_emb, users, items, edge_index):
    eu = edge_index[0]
    ei = edge_index[1]
    zrows = jnp.zeros((ZR, D), _f32)
    pad = ((0, UP - U), (0, 0))
    user_p = jnp.pad(user_emb, pad)
    item_p = jnp.pad(item_emb, pad)
    user_s = user_p + SHIFT
    item_s = item_p + SHIFT

    i_acc1 = _agg(user_s, eu, ei, zrows)
    u_acc1 = _agg(item_s, ei, eu, zrows)
    item1_s, user1_s = _comb(i_acc1, u_acc1, item_p, user_p)

    i_acc2 = _agg(user1_s, eu, ei, zrows)
    u_acc2 = _agg(item1_s, ei, eu, zrows)
    fin_item, fin_user = _comb_fin(i_acc2, u_acc2, item1_s, user1_s,
                                   item_p, user_p)

    lu, li = _batch_gather(fin_user, fin_item, users, items)
    predict = _pred(lu, li)
    return (predict, lu, li)


# speedup vs baseline: 1.0238x; 1.0238x over previous
import jax
import jax.numpy as jnp
from jax import lax
from jax.experimental import pallas as pl
from jax.experimental.pallas import tpu as pltpu
from jax.experimental.pallas import tpu_sc as plsc

U = 10000
UP = 10240
D = 128
E = 320000
B = 4096

NC = 2
NS = 16
NW = NC * NS
PW = E // NW
CH = 40
NCH = PW // CH
NBUF = 5
NG = NCH // NBUF
SBYTES = CH * 128 * 4
RT = UP // NS
ZR = 128
BW = B // NW

_f32 = jnp.float32

_mesh = plsc.VectorSubcoreMesh(
    core_axis_name="c", subcore_axis_name="s", num_cores=NC, num_subcores=NS
)


def _agg_body(tbl, src, dst, zrows, acc_out, *refs):
    ids = (refs[0:NBUF], refs[NBUF:2 * NBUF])
    idd = (refs[2 * NBUF:3 * NBUF], refs[3 * NBUF:4 * NBUF])
    rows, stage, acc_sh, gsem, ssem, isem = refs[4 * NBUF:]
    cid = lax.axis_index("c")
    sid = lax.axis_index("s")
    wid = cid * NS + sid
    r0 = sid * RT
    pltpu.sync_copy(zrows, stage)
    for t in range(RT // ZR):
        pltpu.sync_copy(stage, acc_sh.at[pl.ds(r0 + t * ZR, ZR)])
    plsc.subcore_barrier()

    base = wid * PW

    def fire_idx(g, p):
        for b in range(NBUF):
            off = base + (g * NBUF + b) * CH
            pltpu.async_copy(src.at[pl.ds(off, CH)], ids[p][b], isem)
            pltpu.async_copy(dst.at[pl.ds(off, CH)], idd[p][b], isem)

    fire_idx(0, 0)
    fire_idx(1, 1)

    def super_group(k, carry):
        for p in range(2):
            g = 2 * k + p
            for b in range(NBUF):
                pltpu.make_async_copy(
                    src.at[pl.ds(0, CH)], ids[p][b], isem).wait()
                pltpu.make_async_copy(
                    src.at[pl.ds(0, CH)], idd[p][b], isem).wait()
            @pl.when(g > 0)
            def _():
                for b in range(NBUF):
                    pltpu.make_async_copy(
                        tbl.at[pl.ds(0, CH)], rows.at[b], ssem).wait()
            descs = []
            for b in range(NBUF):
                descs.append(
                    pltpu.async_copy(tbl.at[ids[p][b]], rows.at[b], gsem))
            for b in range(NBUF):
                descs[b].wait()
                pltpu.async_copy(rows.at[b], acc_sh.at[idd[p][b]], ssem,
                                 add=True)
            @pl.when(g + 2 < NG)
            def _():
                fire_idx(g + 2, p)
        return carry

    lax.fori_loop(0, NG // 2, super_group, 0)
    for b in range(NBUF):
        pltpu.make_async_copy(tbl.at[pl.ds(0, CH)], rows.at[b], ssem).wait()
    plsc.subcore_barrier()
    for t in range(RT // ZR):
        sl = pl.ds(r0 + t * ZR, ZR)
        pltpu.sync_copy(acc_sh.at[sl], stage)
        pltpu.sync_copy(stage, acc_out.at[cid, sl])


_agg = pl.kernel(
    _agg_body,
    out_type=jax.ShapeDtypeStruct((NC, UP, D), _f32),
    mesh=_mesh,
    scratch_types=[
        *([pltpu.VMEM((CH,), jnp.int32)] * (4 * NBUF)),
        pltpu.VMEM((NBUF, CH, D), _f32),
        pltpu.VMEM((ZR, D), _f32),
        pltpu.VMEM_SHARED((UP, D), _f32),
        pltpu.SemaphoreType.DMA,
        pltpu.SemaphoreType.DMA,
        pltpu.SemaphoreType.DMA,
    ],
)


def _batch_body(fu, fi, users, items, lu_out, li_out, idx_v, rows_v, sem):
    cid = lax.axis_index("c")
    sid = lax.axis_index("s")
    wid = cid * NS + sid
    b0 = wid * BW
    pltpu.sync_copy(users.at[pl.ds(b0, BW)], idx_v)
    pltpu.async_copy(fu.at[idx_v], rows_v, sem).wait()
    pltpu.sync_copy(rows_v, lu_out.at[pl.ds(b0, BW)])
    pltpu.sync_copy(items.at[pl.ds(b0, BW)], idx_v)
    pltpu.async_copy(fi.at[idx_v], rows_v, sem).wait()
    pltpu.sync_copy(rows_v, li_out.at[pl.ds(b0, BW)])


_batch_gather = pl.kernel(
    _batch_body,
    out_type=(
        jax.ShapeDtypeStruct((B, D), _f32),
        jax.ShapeDtypeStruct((B, D), _f32),
    ),
    mesh=_mesh,
    scratch_types=[
        pltpu.VMEM((BW,), jnp.int32),
        pltpu.VMEM((BW, D), _f32),
        pltpu.SemaphoreType.DMA,
    ],
)


RB = 1024


SHIFT = 8.0


def _norm_terms(acc_ref, self_unshifted):
    t = acc_ref[0] + acc_ref[1]
    d = jnp.round(t[:, 0:1] * (1.0 / SHIFT))
    agg = t - SHIFT * d
    sw = 1.0 - d / (d + 1e-8)
    den = jnp.maximum(d, 1.0)
    return sw * self_unshifted + agg / den


def _comb_kernel(acc_ref, self_ref, out_ref):
    out_ref[...] = _norm_terms(acc_ref, self_ref[...]) + SHIFT


def _comb_fin_kernel(acc_ref, self_ref, base_ref, fin_ref):
    self_u = self_ref[...] - SHIFT
    new = _norm_terms(acc_ref, self_u)
    fin_ref[...] = base_ref[...] + self_u + new


_acc_spec = pl.BlockSpec((NC, RB, D), lambda i: (0, i, 0))
_tbl_spec = pl.BlockSpec((RB, D), lambda i: (i, 0))

_comb = pl.pallas_call(
    _comb_kernel,
    out_shape=jax.ShapeDtypeStruct((UP, D), _f32),
    grid=(UP // RB,),
    in_specs=[_acc_spec, _tbl_spec],
    out_specs=_tbl_spec,
)

_comb_fin = pl.pallas_call(
    _comb_fin_kernel,
    out_shape=jax.ShapeDtypeStruct((UP, D), _f32),
    grid=(UP // RB,),
    in_specs=[_acc_spec, _tbl_spec, _tbl_spec],
    out_specs=_tbl_spec,
)


PB = 512


def _pred_kernel(lu_ref, li_ref, out_ref):
    s = jnp.sum(lu_ref[...] * li_ref[...], axis=1)
    out_ref[...] = jax.nn.sigmoid(s)


_pred = pl.pallas_call(
    _pred_kernel,
    out_shape=jax.ShapeDtypeStruct((B,), _f32),
    grid=(B // PB,),
    in_specs=[
        pl.BlockSpec((PB, D), lambda i: (i, 0)),
        pl.BlockSpec((PB, D), lambda i: (i, 0)),
    ],
    out_specs=pl.BlockSpec((PB,), lambda i: (i,)),
)


def kernel(user_emb, item_emb, users, items, edge_index):
    eu = edge_index[0]
    ei = edge_index[1]
    zrows = jnp.zeros((ZR, D), _f32)
    pad = ((0, UP - U), (0, 0))
    user_p = jnp.pad(user_emb, pad)
    item_p = jnp.pad(item_emb, pad)
    user_s = user_p + SHIFT
    item_s = item_p + SHIFT

    i_acc1 = _agg(user_s, eu, ei, zrows)
    u_acc1 = _agg(item_s, ei, eu, zrows)
    item1_s = _comb(i_acc1, item_p)
    user1_s = _comb(u_acc1, user_p)

    i_acc2 = _agg(user1_s, eu, ei, zrows)
    u_acc2 = _agg(item1_s, ei, eu, zrows)
    fin_item = _comb_fin(i_acc2, item1_s, item_p)
    fin_user = _comb_fin(u_acc2, user1_s, user_p)

    lu, li = _batch_gather(fin_user, fin_item, users, items)
    predict = _pred(lu, li)
    return (predict, lu, li)
